# baseline (speedup 1.0000x reference)
import jax
import jax.numpy as jnp
from jax import lax
from jax.experimental import pallas as pl
from jax.experimental.pallas import tpu as pltpu
from jax.experimental.pallas import tpu_sc as plsc

D_MODEL = 64
SCALE = 8.0
NC, NS, LANES = 2, 16, 16
NW = NC * NS
CHUNK = 128
NBUF = 4
KGRP = D_MODEL // LANES


def _emb_body(x_hbm, table_hbm, out_hbm, idx_v, pidx, ibuf, obuf, *sems):
    gsems = sems[:NBUF]
    ssems = sems[NBUF:]
    rpw = x_hbm.shape[0] // NW
    wid = lax.axis_index("s") * NC + lax.axis_index("c")
    row0 = wid * rpw

    pltpu.sync_copy(x_hbm.at[pl.ds(row0, rpw)], idx_v)

    def fire_gather(cj, b):
        for g in range(CHUNK // LANES):
            iv = idx_v[cj, pl.ds(g * LANES, LANES)]
            pidx[b, pl.ds(g * LANES, LANES)] = iv >> 1
        pltpu.async_copy(table_hbm.at[pidx.at[b]], ibuf.at[b], gsems[b])

    for b in range(NBUF):
        fire_gather(b, b)

    @pl.loop(0, rpw, step=NBUF)
    def _(j):
        for b in range(NBUF):
            cj = j + b
            pltpu.make_async_copy(
                table_hbm.at[pidx.at[b]], ibuf.at[b], gsems[b]
            ).wait()

            @pl.when(cj >= NBUF)
            def _():
                pltpu.make_async_copy(
                    obuf.at[b],
                    out_hbm.at[pl.ds((row0 + cj - NBUF) * CHUNK, CHUNK)],
                    ssems[b],
                ).wait()

            @pl.loop(0, CHUNK, step=LANES)
            def _(r):
                par_v = (idx_v[cj, pl.ds(r, LANES)] & 1) * D_MODEL
                for rr in range(LANES):
                    row = r + rr
                    half = par_v[rr]
                    for k in range(KGRP):
                        obuf[b, row, pl.ds(k * LANES, LANES)] = (
                            ibuf[b, row, pl.ds(half + k * LANES, LANES)] * SCALE
                        )

            pltpu.async_copy(
                obuf.at[b],
                out_hbm.at[pl.ds((row0 + cj) * CHUNK, CHUNK)],
                ssems[b],
            )

            @pl.when(cj + NBUF < rpw)
            def _():
                fire_gather(cj + NBUF, b)

    for b in range(NBUF):
        pltpu.make_async_copy(
            obuf.at[b], out_hbm.at[pl.ds(row0 * CHUNK, CHUNK)], ssems[b]
        ).wait()


def kernel(x, table):
    b0, b1 = x.shape
    total = b0 * b1
    xf = x.reshape(total // CHUNK, CHUNK)
    t2 = jnp.concatenate([table[0::2], table[1::2]], axis=1)
    run = pl.kernel(
        _emb_body,
        out_type=jax.ShapeDtypeStruct((total, D_MODEL), jnp.float32),
        mesh=plsc.VectorSubcoreMesh(core_axis_name="c", subcore_axis_name="s"),
        scratch_types=[
            pltpu.VMEM((total // CHUNK // NW, CHUNK), jnp.int32),
            pltpu.VMEM((NBUF, CHUNK), jnp.int32),
            pltpu.VMEM((NBUF, CHUNK, 2 * D_MODEL), jnp.float32),
            pltpu.VMEM((NBUF, CHUNK, D_MODEL), jnp.float32),
        ]
        + [pltpu.SemaphoreType.DMA] * (2 * NBUF),
        compiler_params=pltpu.CompilerParams(use_tc_tiling_on_sc=False),
    )
    out = run(xf, t2)
    return out.reshape(b0, b1, D_MODEL)


# speedup vs baseline: 7.9464x; 7.9464x over previous
import jax
import jax.numpy as jnp
from jax import lax
from jax.experimental import pallas as pl
from jax.experimental.pallas import tpu as pltpu
from jax.experimental.pallas import tpu_sc as plsc

D_MODEL = 64
SCALE = 8.0
NC, NS, LANES = 2, 16, 16
NW = NC * NS
CHUNK = 128
NBUF = 4
KGRP = D_MODEL // LANES


def _emb_body(x_hbm, table_hbm, out_hbm, idx_v, ibuf, obuf, *sems):
    gsems = sems[:NBUF]
    ssems = sems[NBUF:]
    rpw = x_hbm.shape[0] // NW
    wid = lax.axis_index("s") * NC + lax.axis_index("c")
    row0 = wid * rpw

    pltpu.sync_copy(x_hbm.at[pl.ds(row0, rpw)], idx_v)

    def fire_gather(cj, b):
        pltpu.async_copy(table_hbm.at[idx_v.at[cj]], ibuf.at[b], gsems[b])

    for b in range(NBUF):
        fire_gather(b, b)

    @pl.loop(0, rpw, step=NBUF)
    def _(j):
        for b in range(NBUF):
            cj = j + b
            pltpu.make_async_copy(
                table_hbm.at[idx_v.at[cj]], ibuf.at[b], gsems[b]
            ).wait()

            @pl.when(cj >= NBUF)
            def _():
                pltpu.make_async_copy(
                    obuf.at[b],
                    out_hbm.at[pl.ds((row0 + cj - NBUF) * CHUNK, CHUNK)],
                    ssems[b],
                ).wait()

            @pl.loop(0, CHUNK, step=LANES)
            def _(r):
                for rr in range(LANES):
                    row = r + rr
                    for k in range(KGRP):
                        sl = pl.ds(k * LANES, LANES)
                        obuf[b, row, sl] = ibuf[b, row, sl] * SCALE

            pltpu.async_copy(
                obuf.at[b],
                out_hbm.at[pl.ds((row0 + cj) * CHUNK, CHUNK)],
                ssems[b],
            )

            @pl.when(cj + NBUF < rpw)
            def _():
                fire_gather(cj + NBUF, b)

    for b in range(NBUF):
        pltpu.make_async_copy(
            obuf.at[b], out_hbm.at[pl.ds(row0 * CHUNK, CHUNK)], ssems[b]
        ).wait()


def kernel(x, table):
    b0, b1 = x.shape
    total = b0 * b1
    xf = x.reshape(total // CHUNK, CHUNK)
    run = pl.kernel(
        _emb_body,
        out_type=jax.ShapeDtypeStruct((total, D_MODEL), jnp.float32),
        mesh=plsc.VectorSubcoreMesh(core_axis_name="c", subcore_axis_name="s"),
        scratch_types=[
            pltpu.VMEM((total // CHUNK // NW, CHUNK), jnp.int32),
            pltpu.VMEM((NBUF, CHUNK, D_MODEL), jnp.float32),
            pltpu.VMEM((NBUF, CHUNK, D_MODEL), jnp.float32),
        ]
        + [pltpu.SemaphoreType.DMA] * (2 * NBUF),
        compiler_params=pltpu.CompilerParams(use_tc_tiling_on_sc=False),
    )
    out = run(xf, table)
    return out.reshape(b0, b1, D_MODEL)
